# baseline (speedup 1.0000x reference)
import functools

import jax
import jax.numpy as jnp
from jax import lax
from jax.experimental import pallas as pl
from jax.experimental.pallas import tpu as pltpu
from jax.experimental.pallas import tpu_sc as plsc

EPS = 1e-5
KNN = 20
_CP = 16
_HI = lax.Precision.HIGHEST
_DEF = lax.Precision.DEFAULT


def _dot_t(a, b, prec):
    return lax.dot_general(a, b, (((1,), (1,)), ((), ())), precision=prec,
                           preferred_element_type=jnp.float32)


def _knn_body(N, RB, K, xr_ref, xb_ref, idx_ref):
    b = pl.program_id(0)
    rb = pl.program_id(1)
    xr = xr_ref[...]
    xb = xb_ref[...]
    t1 = -2.0 * _dot_t(xr, xb, _DEF)
    xxr = jnp.sum(xr * xr, axis=1)
    xx = jnp.sum(xb * xb, axis=1)
    P = (-xxr[:, None] - t1) - xx[None, :]
    iota = lax.broadcasted_iota(jnp.int32, (RB, N), 1)
    rows = lax.broadcasted_iota(jnp.int32, (RB, N), 0) + rb * RB
    off = b * N
    idx_ref[0, 0, :] = rows[:, 0] + off
    P = jnp.where(iota == rows, -1e30, P)
    for t in range(1, K):
        m = jnp.max(P, axis=1)
        cand = jnp.where(P >= m[:, None], iota, N)
        a = jnp.min(cand, axis=1)
        idx_ref[0, t, :] = a + off
        P = jnp.where(iota == a[:, None], -1e30, P)


def _knn(x_t, B, N):
    C = x_t.shape[1]
    RB = 256
    nrb = N // RB
    return pl.pallas_call(
        functools.partial(_knn_body, N, RB, KNN),
        grid=(B, nrb),
        in_specs=[
            pl.BlockSpec((RB, C), lambda b, r: (b * nrb + r, 0)),
            pl.BlockSpec((N, C), lambda b, r: (b, 0)),
        ],
        out_specs=pl.BlockSpec((1, KNN, RB), lambda b, r: (b, 0, r)),
        out_shape=jax.ShapeDtypeStruct((B, KNN, N), jnp.int32),
    )(x_t, x_t)


def _sc_gather(table, idx, B, N):
    R = B * N
    TD = table.shape[1]
    K = KNN
    info = plsc.get_sparse_core_info()
    NC, NS = info.num_cores, info.num_subcores
    NW = NC * NS
    P = R // NW
    CP = _CP
    NCH = P // CP

    mesh = plsc.VectorSubcoreMesh(core_axis_name="c", subcore_axis_name="s")

    @functools.partial(
        pl.kernel, mesh=mesh,
        out_type=jax.ShapeDtypeStruct((R * K, TD), jnp.float32),
        scratch_types=[
            pltpu.VMEM((K, P), jnp.int32),
            pltpu.VMEM((K * CP, TD), jnp.float32),
            pltpu.VMEM((K * CP, TD), jnp.float32),
            pltpu.SemaphoreType.DMA,
            pltpu.SemaphoreType.DMA,
        ],
    )
    def k(table_hbm, idx_hbm, e_hbm, idx_v, rows_a, rows_b, sem_a, sem_b):
        wid = lax.axis_index("s") * NC + lax.axis_index("c")
        base = wid * P
        b = base // N
        nb = base % N
        pltpu.sync_copy(idx_hbm.at[b, :, pl.ds(nb, P)], idx_v)
        bufs = [(rows_a, sem_a), (rows_b, sem_b)]

        def fire(c, buf, sem):
            return [pltpu.async_copy(
                table_hbm.at[idx_v.at[t, pl.ds(c * CP, CP)]],
                buf.at[pl.ds(t * CP, CP)], sem) for t in range(K)]

        pend = fire(0, *bufs[0])
        for c in range(NCH):
            nxt = fire(c + 1, *bufs[(c + 1) % 2]) if c + 1 < NCH else []
            for h in pend:
                h.wait()
            buf = bufs[c % 2][0]
            row0 = (wid * NCH + c) * K * CP
            pltpu.sync_copy(buf, e_hbm.at[pl.ds(row0, K * CP)])
            pend = nxt

    return k(table, idx)


def _edge_pool_body(NCH, CP, K, C, e_ref, bit_ref, x_ref, wa_ref, wb_ref,
                    mx_ref, sm_ref, ss_ref):
    ev = e_ref[...]
    G = jnp.where(bit_ref[...] > 0, ev[:, C:2 * C], ev[:, :C])
    xi = x_ref[...]
    xi4 = xi.reshape(NCH, 1, CP, C)
    xib = jnp.broadcast_to(xi4, (NCH, K, CP, C)).reshape(NCH * K * CP, C)
    diff = G - xib
    v = _dot_t(diff, wa_ref[...], _DEF)
    zi = _dot_t(xi, wb_ref[...], _DEF)
    O = v.shape[1]
    v4 = v.reshape(NCH, K, CP, O) + zi.reshape(NCH, 1, CP, O)
    mx_ref[...] = jnp.max(v4, axis=1).reshape(NCH * CP, O)
    sm_ref[...] = jnp.sum(v4, axis=1).reshape(NCH * CP, O)
    ss_ref[...] = jnp.sum(v4 * v4, axis=1).reshape(NCH * CP, O)


def _edge_pool(E, bits, x_t, wa, wb, B, N):
    R = B * N
    C = x_t.shape[1]
    O = wa.shape[0]
    K = KNN
    NW = 32
    P = R // NW
    CP = _CP
    NCH = P // CP
    out_spec = pl.BlockSpec((P, O), lambda w: (w, 0))
    out_sh = jax.ShapeDtypeStruct((R, O), jnp.float32)
    return pl.pallas_call(
        functools.partial(_edge_pool_body, NCH, CP, K, C),
        grid=(NW,),
        in_specs=[
            pl.BlockSpec((P * K, E.shape[1]), lambda w: (w, 0)),
            pl.BlockSpec((P * K, 1), lambda w: (w, 0)),
            pl.BlockSpec((P, C), lambda w: (w, 0)),
            pl.BlockSpec((O, C), lambda w: (0, 0)),
            pl.BlockSpec((O, C), lambda w: (0, 0)),
        ],
        out_specs=[out_spec, out_spec, out_spec],
        out_shape=[out_sh, out_sh, out_sh],
    )(E, bits, x_t, wa, wb)


def _combine_body(R, K, mx_ref, sm_ref, ss_ref, g_ref, b_ref, o_ref):
    cnt = float(R * K)
    m = jnp.sum(sm_ref[...], axis=0) / cnt
    e2 = jnp.sum(ss_ref[...], axis=0) / cnt
    var = e2 - m * m
    scale = g_ref[0] * lax.rsqrt(var + EPS)
    v = (mx_ref[...] - m[None, :]) * scale[None, :] + b_ref[0][None, :]
    o_ref[...] = jnp.where(v > 0, v, 0.2 * v)


def _combine(mx, sm, ss, g, b):
    R, D = mx.shape
    return pl.pallas_call(
        functools.partial(_combine_body, R, KNN),
        out_shape=jax.ShapeDtypeStruct((R, D), jnp.float32),
    )(mx, sm, ss, g.reshape(1, D), b.reshape(1, D))


def _sigmoid(x):
    return 1.0 / (1.0 + jnp.exp(-x))


def _head_a_body(N, x1_ref, x2_ref, w3a_ref, w3b_ref, w5a0_ref, w5a1_ref,
                 w5a2_ref, w5b0_ref, w5b1_ref, w5b2_ref, a_ref, c_ref,
                 am_ref, cm_ref):
    x1 = x1_ref[...]
    x2 = x2_ref[...]
    a = _dot_t(x1, w3a_ref[...], _HI) + _dot_t(x2, w3b_ref[...], _HI)

    def shift_prev(u):
        zr = jnp.zeros((1, u.shape[1]), jnp.float32)
        return jnp.concatenate([zr, u[:-1, :]], axis=0)

    def shift_next(u):
        zr = jnp.zeros((1, u.shape[1]), jnp.float32)
        return jnp.concatenate([u[1:, :], zr], axis=0)

    c = (_dot_t(shift_prev(x1), w5a0_ref[...], _HI)
         + _dot_t(x1, w5a1_ref[...], _HI)
         + _dot_t(shift_next(x1), w5a2_ref[...], _HI)
         + _dot_t(shift_prev(x2), w5b0_ref[...], _HI)
         + _dot_t(x2, w5b1_ref[...], _HI)
         + _dot_t(shift_next(x2), w5b2_ref[...], _HI))
    a_ref[0] = a
    c_ref[0] = c
    am_ref[0] = jnp.mean(a, axis=0, keepdims=True)
    cm_ref[0] = jnp.mean(c, axis=0, keepdims=True)


def _head_b_body(g3_ref, b3_ref, sw1_ref, sw2_ref, a_ref, c_ref, am_ref,
                 cm_ref, o_ref):
    def se_scale(ym):
        y2 = jnp.maximum(_dot_t(ym, sw1_ref[...], _HI), 0.0)
        return _sigmoid(_dot_t(y2, sw2_ref[...], _HI))

    ya = se_scale(am_ref[:, 0, :])
    yc = se_scale(cm_ref[:, 0, :])
    s = a_ref[...] * ya[:, None, :] + c_ref[...] * yc[:, None, :]
    m3 = jnp.mean(s, axis=(0, 1))
    v3 = jnp.mean(s * s, axis=(0, 1)) - m3 * m3
    sc3 = g3_ref[0] * lax.rsqrt(v3 + EPS)
    u = (s - m3[None, None, :]) * sc3[None, None, :] + b3_ref[0][None, None, :]
    o_ref[...] = jnp.where(u > 0, u, 0.2 * u)


def _head(x1t, x2t, W3, W5, sw1, sw2, g3, b3, B, N):
    D1 = x1t.shape[1]
    D2 = x2t.shape[1]
    O = W3.shape[0]
    w3a, w3b = W3[:, :D1], W3[:, D1:]
    w5 = [(W5[:, :D1, t], W5[:, D1:, t]) for t in range(3)]
    wspec1 = pl.BlockSpec((O, D1), lambda b: (0, 0))
    wspec2 = pl.BlockSpec((O, D2), lambda b: (0, 0))
    a_pre, c_pre, am, cm = pl.pallas_call(
        functools.partial(_head_a_body, N),
        grid=(B,),
        in_specs=[
            pl.BlockSpec((N, D1), lambda b: (b, 0)),
            pl.BlockSpec((N, D2), lambda b: (b, 0)),
            wspec1, wspec2, wspec1, wspec1, wspec1, wspec2, wspec2, wspec2,
        ],
        out_specs=[
            pl.BlockSpec((1, N, O), lambda b: (b, 0, 0)),
            pl.BlockSpec((1, N, O), lambda b: (b, 0, 0)),
            pl.BlockSpec((1, 1, O), lambda b: (b, 0, 0)),
            pl.BlockSpec((1, 1, O), lambda b: (b, 0, 0)),
        ],
        out_shape=[
            jax.ShapeDtypeStruct((B, N, O), jnp.float32),
            jax.ShapeDtypeStruct((B, N, O), jnp.float32),
            jax.ShapeDtypeStruct((B, 1, O), jnp.float32),
            jax.ShapeDtypeStruct((B, 1, O), jnp.float32),
        ],
    )(x1t, x2t, w3a, w3b, w5[0][0], w5[1][0], w5[2][0], w5[0][1], w5[1][1],
      w5[2][1])
    out = pl.pallas_call(
        _head_b_body,
        out_shape=jax.ShapeDtypeStruct((B, N, O), jnp.float32),
    )(g3.reshape(1, O), b3.reshape(1, O), sw1, sw2, a_pre, c_pre, am, cm)
    return out


def _edge_layer(x_t, W, g, b, B, N):
    R, C = x_t.shape
    K = KNN
    wa, wb = W[:, :C], W[:, C:]
    idx = _knn(x_t, B, N)
    xpair = x_t.reshape(R // 2, 2 * C)
    idxh = idx >> 1
    NW, CP = 32, _CP
    NCH = (R // NW) // CP
    bits = ((idx & 1).transpose(0, 2, 1).reshape(NW, NCH, CP, K)
            .transpose(0, 1, 3, 2).reshape(R * K, 1))
    E = _sc_gather(xpair, idxh, B, N)
    mx, sm, ss = _edge_pool(E, bits, x_t, wa, wb, B, N)
    return _combine(mx, sm, ss, g, b)


def kernel(x, xyz, W1, g1, b1, W2, g2, b2, W3, W5, g3, b3, sw1, sw2):
    B, C, N = x.shape
    x_t = jnp.transpose(x, (0, 2, 1)).reshape(B * N, C)
    x1t = _edge_layer(x_t, W1, g1, b1, B, N)
    x2t = _edge_layer(x1t, W2, g2, b2, B, N)
    out_t = _head(x1t, x2t, W3, W5, sw1, sw2, g3, b3, B, N)
    out = jnp.transpose(out_t, (0, 2, 1))
    return out, xyz


# speedup vs baseline: 1.1362x; 1.1362x over previous
import functools

import jax
import jax.numpy as jnp
from jax import lax
from jax.experimental import pallas as pl
from jax.experimental.pallas import tpu as pltpu
from jax.experimental.pallas import tpu_sc as plsc

EPS = 1e-5
KNN = 20
_CP = 16
_HI = lax.Precision.HIGHEST
_DEF = lax.Precision.DEFAULT


def _dot_t(a, b, prec):
    return lax.dot_general(a, b, (((1,), (1,)), ((), ())), precision=prec,
                           preferred_element_type=jnp.float32)


def _knn_body(N, RB, K, B0, xr_ref, xb_ref, idx_ref):
    b = pl.program_id(0) + B0
    rb = pl.program_id(1)
    xr = xr_ref[...]
    xb = xb_ref[...]
    t1 = -2.0 * _dot_t(xr, xb, _DEF)
    xxr = jnp.sum(xr * xr, axis=1)
    xx = jnp.sum(xb * xb, axis=1)
    P = (-xxr[:, None] - t1) - xx[None, :]
    iota = lax.broadcasted_iota(jnp.int32, (RB, N), 1)
    rows = lax.broadcasted_iota(jnp.int32, (RB, N), 0) + rb * RB
    off = b * N
    idx_ref[0, 0, :] = rows[:, 0] + off
    P = jnp.where(iota == rows, -1e30, P)
    for t in range(1, K):
        m = jnp.max(P, axis=1)
        cand = jnp.where(P >= m[:, None], iota, N)
        a = jnp.min(cand, axis=1)
        idx_ref[0, t, :] = a + off
        P = jnp.where(iota == a[:, None], -1e30, P)


def _knn(x_t, N, b0, nb):
    C = x_t.shape[1]
    RB = 256
    nrb = N // RB
    return pl.pallas_call(
        functools.partial(_knn_body, N, RB, KNN, b0),
        grid=(nb, nrb),
        in_specs=[
            pl.BlockSpec((RB, C), lambda b, r: ((b0 + b) * nrb + r, 0)),
            pl.BlockSpec((N, C), lambda b, r: (b0 + b, 0)),
        ],
        out_specs=pl.BlockSpec((1, KNN, RB), lambda b, r: (b, 0, r)),
        out_shape=jax.ShapeDtypeStruct((nb, KNN, N), jnp.int32),
    )(x_t, x_t)


def _sc_gather(table, idx, N):
    R = idx.shape[0] * N
    TD = table.shape[1]
    K = KNN
    info = plsc.get_sparse_core_info()
    NC, NS = info.num_cores, info.num_subcores
    NW = NC * NS
    P = R // NW
    CP = _CP
    NCH = P // CP

    mesh = plsc.VectorSubcoreMesh(core_axis_name="c", subcore_axis_name="s")

    @functools.partial(
        pl.kernel, mesh=mesh,
        out_type=jax.ShapeDtypeStruct((R * K, TD), jnp.float32),
        scratch_types=[
            pltpu.VMEM((K, P), jnp.int32),
            pltpu.VMEM((K * CP, TD), jnp.float32),
            pltpu.VMEM((K * CP, TD), jnp.float32),
            pltpu.SemaphoreType.DMA,
            pltpu.SemaphoreType.DMA,
        ],
    )
    def k(table_hbm, idx_hbm, e_hbm, idx_v, rows_a, rows_b, sem_a, sem_b):
        wid = lax.axis_index("s") * NC + lax.axis_index("c")
        base = wid * P
        b = base // N
        nb = base % N
        pltpu.sync_copy(idx_hbm.at[b, :, pl.ds(nb, P)], idx_v)
        bufs = [(rows_a, sem_a), (rows_b, sem_b)]

        def fire(c, buf, sem):
            return [pltpu.async_copy(
                table_hbm.at[idx_v.at[t, pl.ds(c * CP, CP)]],
                buf.at[pl.ds(t * CP, CP)], sem) for t in range(K)]

        pend = fire(0, *bufs[0])
        for c in range(NCH):
            nxt = fire(c + 1, *bufs[(c + 1) % 2]) if c + 1 < NCH else []
            for h in pend:
                h.wait()
            buf = bufs[c % 2][0]
            row0 = (wid * NCH + c) * K * CP
            pltpu.sync_copy(buf, e_hbm.at[pl.ds(row0, K * CP)])
            pend = nxt

    return k(table, idx)


def _edge_pool_body(NCH, CP, K, C, e_ref, x_ref, wa_ref, wb_ref,
                    mx_ref, sm_ref, ss_ref):
    G = e_ref[...][:, :C]
    xi = x_ref[...]
    xi4 = xi.reshape(NCH, 1, CP, C)
    xib = jnp.broadcast_to(xi4, (NCH, K, CP, C)).reshape(NCH * K * CP, C)
    diff = G - xib
    v = _dot_t(diff, wa_ref[...], _DEF)
    zi = _dot_t(xi, wb_ref[...], _DEF)
    O = v.shape[1]
    v4 = v.reshape(NCH, K, CP, O) + zi.reshape(NCH, 1, CP, O)
    mx_ref[...] = jnp.max(v4, axis=1).reshape(NCH * CP, O)
    sm_ref[...] = jnp.sum(v4, axis=1).reshape(NCH * CP, O)
    ss_ref[...] = jnp.sum(v4 * v4, axis=1).reshape(NCH * CP, O)


def _edge_pool(E, x_t, wa, wb):
    R = x_t.shape[0]
    C = x_t.shape[1]
    O = wa.shape[0]
    K = KNN
    NW = 32
    P = R // NW
    CP = _CP
    NCH = P // CP
    out_spec = pl.BlockSpec((P, O), lambda w: (w, 0))
    out_sh = jax.ShapeDtypeStruct((R, O), jnp.float32)
    return pl.pallas_call(
        functools.partial(_edge_pool_body, NCH, CP, K, C),
        grid=(NW,),
        in_specs=[
            pl.BlockSpec((P * K, E.shape[1]), lambda w: (w, 0)),
            pl.BlockSpec((P, C), lambda w: (w, 0)),
            pl.BlockSpec((O, C), lambda w: (0, 0)),
            pl.BlockSpec((O, C), lambda w: (0, 0)),
        ],
        out_specs=[out_spec, out_spec, out_spec],
        out_shape=[out_sh, out_sh, out_sh],
    )(E, x_t, wa, wb)


def _combine_body(R, K, mx_ref, sm_ref, ss_ref, g_ref, b_ref, o_ref):
    cnt = float(R * K)
    m = jnp.sum(sm_ref[...], axis=0) / cnt
    e2 = jnp.sum(ss_ref[...], axis=0) / cnt
    var = e2 - m * m
    scale = g_ref[0] * lax.rsqrt(var + EPS)
    v = (mx_ref[...] - m[None, :]) * scale[None, :] + b_ref[0][None, :]
    o_ref[...] = jnp.where(v > 0, v, 0.2 * v)


def _combine(mx, sm, ss, g, b):
    R, D = mx.shape
    return pl.pallas_call(
        functools.partial(_combine_body, R, KNN),
        out_shape=jax.ShapeDtypeStruct((R, D), jnp.float32),
    )(mx, sm, ss, g.reshape(1, D), b.reshape(1, D))


def _sigmoid(x):
    return 1.0 / (1.0 + jnp.exp(-x))


def _head_a_body(N, x1_ref, x2_ref, w3a_ref, w3b_ref, w5a0_ref, w5a1_ref,
                 w5a2_ref, w5b0_ref, w5b1_ref, w5b2_ref, a_ref, c_ref,
                 am_ref, cm_ref):
    x1 = x1_ref[...]
    x2 = x2_ref[...]
    a = _dot_t(x1, w3a_ref[...], _HI) + _dot_t(x2, w3b_ref[...], _HI)

    def shift_prev(u):
        zr = jnp.zeros((1, u.shape[1]), jnp.float32)
        return jnp.concatenate([zr, u[:-1, :]], axis=0)

    def shift_next(u):
        zr = jnp.zeros((1, u.shape[1]), jnp.float32)
        return jnp.concatenate([u[1:, :], zr], axis=0)

    c = (_dot_t(shift_prev(x1), w5a0_ref[...], _HI)
         + _dot_t(x1, w5a1_ref[...], _HI)
         + _dot_t(shift_next(x1), w5a2_ref[...], _HI)
         + _dot_t(shift_prev(x2), w5b0_ref[...], _HI)
         + _dot_t(x2, w5b1_ref[...], _HI)
         + _dot_t(shift_next(x2), w5b2_ref[...], _HI))
    a_ref[0] = a
    c_ref[0] = c
    am_ref[0] = jnp.mean(a, axis=0, keepdims=True)
    cm_ref[0] = jnp.mean(c, axis=0, keepdims=True)


def _head_b_body(g3_ref, b3_ref, sw1_ref, sw2_ref, a_ref, c_ref, am_ref,
                 cm_ref, o_ref):
    def se_scale(ym):
        y2 = jnp.maximum(_dot_t(ym, sw1_ref[...], _HI), 0.0)
        return _sigmoid(_dot_t(y2, sw2_ref[...], _HI))

    ya = se_scale(am_ref[:, 0, :])
    yc = se_scale(cm_ref[:, 0, :])
    s = a_ref[...] * ya[:, None, :] + c_ref[...] * yc[:, None, :]
    m3 = jnp.mean(s, axis=(0, 1))
    v3 = jnp.mean(s * s, axis=(0, 1)) - m3 * m3
    sc3 = g3_ref[0] * lax.rsqrt(v3 + EPS)
    u = (s - m3[None, None, :]) * sc3[None, None, :] + b3_ref[0][None, None, :]
    o_ref[...] = jnp.where(u > 0, u, 0.2 * u)


def _head(x1t, x2t, W3, W5, sw1, sw2, g3, b3, B, N):
    D1 = x1t.shape[1]
    D2 = x2t.shape[1]
    O = W3.shape[0]
    w3a, w3b = W3[:, :D1], W3[:, D1:]
    w5 = [(W5[:, :D1, t], W5[:, D1:, t]) for t in range(3)]
    wspec1 = pl.BlockSpec((O, D1), lambda b: (0, 0))
    wspec2 = pl.BlockSpec((O, D2), lambda b: (0, 0))
    a_pre, c_pre, am, cm = pl.pallas_call(
        functools.partial(_head_a_body, N),
        grid=(B,),
        in_specs=[
            pl.BlockSpec((N, D1), lambda b: (b, 0)),
            pl.BlockSpec((N, D2), lambda b: (b, 0)),
            wspec1, wspec2, wspec1, wspec1, wspec1, wspec2, wspec2, wspec2,
        ],
        out_specs=[
            pl.BlockSpec((1, N, O), lambda b: (b, 0, 0)),
            pl.BlockSpec((1, N, O), lambda b: (b, 0, 0)),
            pl.BlockSpec((1, 1, O), lambda b: (b, 0, 0)),
            pl.BlockSpec((1, 1, O), lambda b: (b, 0, 0)),
        ],
        out_shape=[
            jax.ShapeDtypeStruct((B, N, O), jnp.float32),
            jax.ShapeDtypeStruct((B, N, O), jnp.float32),
            jax.ShapeDtypeStruct((B, 1, O), jnp.float32),
            jax.ShapeDtypeStruct((B, 1, O), jnp.float32),
        ],
    )(x1t, x2t, w3a, w3b, w5[0][0], w5[1][0], w5[2][0], w5[0][1], w5[1][1],
      w5[2][1])
    out = pl.pallas_call(
        _head_b_body,
        out_shape=jax.ShapeDtypeStruct((B, N, O), jnp.float32),
    )(g3.reshape(1, O), b3.reshape(1, O), sw1, sw2, a_pre, c_pre, am, cm)
    return out


def _edge_layer(x_t, W, g, b, B, N):
    R, C = x_t.shape
    wa, wb = W[:, :C], W[:, C:]
    xpad = jnp.pad(x_t, ((0, 0), (0, 128 - C))) if C < 128 else x_t
    parts = []
    nb = B // 2
    for h in range(2):
        idx = _knn(x_t, N, h * nb, nb)
        E = _sc_gather(xpad, idx, N)
        xp = x_t[h * (R // 2):(h + 1) * (R // 2)]
        parts.append(_edge_pool(E, xp, wa, wb))
    mx, sm, ss = (jnp.concatenate([parts[0][i], parts[1][i]])
                  for i in range(3))
    return _combine(mx, sm, ss, g, b)


def kernel(x, xyz, W1, g1, b1, W2, g2, b2, W3, W5, g3, b3, sw1, sw2):
    B, C, N = x.shape
    x_t = jnp.transpose(x, (0, 2, 1)).reshape(B * N, C)
    x1t = _edge_layer(x_t, W1, g1, b1, B, N)
    x2t = _edge_layer(x1t, W2, g2, b2, B, N)
    out_t = _head(x1t, x2t, W3, W5, sw1, sw2, g3, b3, B, N)
    out = jnp.transpose(out_t, (0, 2, 1))
    return out, xyz


# speedup vs baseline: 1.1475x; 1.0099x over previous
import functools

import jax
import jax.numpy as jnp
from jax import lax
from jax.experimental import pallas as pl
from jax.experimental.pallas import tpu as pltpu
from jax.experimental.pallas import tpu_sc as plsc

EPS = 1e-5
KNN = 20
_CP = 16
_HI = lax.Precision.HIGHEST
_DEF = lax.Precision.DEFAULT


def _dot_t(a, b, prec):
    return lax.dot_general(a, b, (((1,), (1,)), ((), ())), precision=prec,
                           preferred_element_type=jnp.float32)


def _knn_body(N, RB, K, B0, xr_ref, xb_ref, idx_ref):
    b = pl.program_id(0) + B0
    rb = pl.program_id(1)
    xr = xr_ref[...]
    xb = xb_ref[...]
    t1 = -2.0 * _dot_t(xr, xb, _DEF)
    xxr = jnp.sum(xr * xr, axis=1)
    xx = jnp.sum(xb * xb, axis=1)
    P = (-xxr[:, None] - t1) - xx[None, :]
    iota = lax.broadcasted_iota(jnp.int32, (RB, N), 1)
    rows = lax.broadcasted_iota(jnp.int32, (RB, N), 0) + rb * RB
    off = b * N
    idx_ref[0, 0, :] = rows[:, 0] + off
    P = jnp.where(iota == rows, -1e30, P)
    for t in range(1, K):
        m = jnp.max(P, axis=1)
        cand = jnp.where(P >= m[:, None], iota, N)
        a = jnp.min(cand, axis=1)
        idx_ref[0, t, :] = a + off
        P = jnp.where(iota == a[:, None], -1e30, P)


def _knn(x_t, N, b0, nb):
    C = x_t.shape[1]
    RB = 256
    nrb = N // RB
    return pl.pallas_call(
        functools.partial(_knn_body, N, RB, KNN, b0),
        grid=(nb, nrb),
        in_specs=[
            pl.BlockSpec((RB, C), lambda b, r: ((b0 + b) * nrb + r, 0)),
            pl.BlockSpec((N, C), lambda b, r: (b0 + b, 0)),
        ],
        out_specs=pl.BlockSpec((1, KNN, RB), lambda b, r: (b, 0, r)),
        out_shape=jax.ShapeDtypeStruct((nb, KNN, N), jnp.int32),
    )(x_t, x_t)


def _sc_gather(table, idx, N):
    R = idx.shape[0] * N
    TD = table.shape[1]
    K = KNN
    info = plsc.get_sparse_core_info()
    NC, NS = info.num_cores, info.num_subcores
    NW = NC * NS
    P = R // NW
    CP = _CP
    NCH = P // CP

    mesh = plsc.VectorSubcoreMesh(core_axis_name="c", subcore_axis_name="s")

    @functools.partial(
        pl.kernel, mesh=mesh,
        out_type=jax.ShapeDtypeStruct((R * K, TD), jnp.float32),
        scratch_types=[
            pltpu.VMEM((K, P), jnp.int32),
            pltpu.VMEM((K * CP, TD), jnp.float32),
            pltpu.VMEM((K * CP, TD), jnp.float32),
            pltpu.SemaphoreType.DMA,
            pltpu.SemaphoreType.DMA,
        ],
    )
    def k(table_hbm, idx_hbm, e_hbm, idx_v, rows_a, rows_b, sem_a, sem_b):
        wid = lax.axis_index("s") * NC + lax.axis_index("c")
        base = wid * P
        b = base // N
        nb = base % N
        pltpu.sync_copy(idx_hbm.at[b, :, pl.ds(nb, P)], idx_v)
        bufs = [(rows_a, sem_a), (rows_b, sem_b)]

        def fire(c, buf, sem):
            return [pltpu.async_copy(
                table_hbm.at[idx_v.at[t, pl.ds(c * CP, CP)]],
                buf.at[pl.ds(t * CP, CP)], sem) for t in range(K)]

        pend = fire(0, *bufs[0])
        for c in range(NCH):
            nxt = fire(c + 1, *bufs[(c + 1) % 2]) if c + 1 < NCH else []
            for h in pend:
                h.wait()
            buf = bufs[c % 2][0]
            row0 = (wid * NCH + c) * K * CP
            pltpu.sync_copy(buf, e_hbm.at[pl.ds(row0, K * CP)])
            pend = nxt

    return k(table, idx)


def _edge_pool_body(NCH, CP, K, C, e_ref, x_ref, wa_ref, wb_ref,
                    mx_ref, sm_ref, ss_ref):
    G = e_ref[...]
    xi = x_ref[...]
    xi4 = xi.reshape(NCH, 1, CP, C)
    xib = jnp.broadcast_to(xi4, (NCH, K, CP, C)).reshape(NCH * K * CP, C)
    diff = G - xib
    v = _dot_t(diff, wa_ref[...], _DEF)
    zi = _dot_t(xi, wb_ref[...], _DEF)
    O = v.shape[1]
    v4 = v.reshape(NCH, K, CP, O) + zi.reshape(NCH, 1, CP, O)
    mx_ref[...] = jnp.max(v4, axis=1).reshape(NCH * CP, O)
    sm_ref[...] = jnp.sum(v4, axis=1).reshape(NCH * CP, O)
    ss_ref[...] = jnp.sum(v4 * v4, axis=1).reshape(NCH * CP, O)


def _edge_pool(E, x_t, wa, wb):
    R, C = x_t.shape
    O = wa.shape[0]
    K = KNN
    NW = 32
    P = R // NW
    CP = _CP
    NCH = P // CP
    out_spec = pl.BlockSpec((P, O), lambda w: (w, 0))
    out_sh = jax.ShapeDtypeStruct((R, O), jnp.float32)
    return pl.pallas_call(
        functools.partial(_edge_pool_body, NCH, CP, K, C),
        grid=(NW,),
        in_specs=[
            pl.BlockSpec((P * K, E.shape[1]), lambda w: (w, 0)),
            pl.BlockSpec((P, C), lambda w: (w, 0)),
            pl.BlockSpec((O, C), lambda w: (0, 0)),
            pl.BlockSpec((O, C), lambda w: (0, 0)),
        ],
        out_specs=[out_spec, out_spec, out_spec],
        out_shape=[out_sh, out_sh, out_sh],
    )(E, x_t, wa, wb)


def _combine_body(R, K, mx_ref, sm_ref, ss_ref, g_ref, b_ref, o_ref):
    cnt = float(R * K)
    m = jnp.sum(sm_ref[...], axis=0) / cnt
    e2 = jnp.sum(ss_ref[...], axis=0) / cnt
    var = e2 - m * m
    scale = g_ref[0] * lax.rsqrt(var + EPS)
    v = (mx_ref[...] - m[None, :]) * scale[None, :] + b_ref[0][None, :]
    o_ref[...] = jnp.where(v > 0, v, 0.2 * v)


def _combine(mx, sm, ss, g, b):
    R, D = mx.shape
    return pl.pallas_call(
        functools.partial(_combine_body, R, KNN),
        out_shape=jax.ShapeDtypeStruct((R, D), jnp.float32),
    )(mx, sm, ss, g.reshape(1, D), b.reshape(1, D))


def _sigmoid(x):
    return 1.0 / (1.0 + jnp.exp(-x))


def _head_a_body(N, x1_ref, x2_ref, w3a_ref, w3b_ref, w5a0_ref, w5a1_ref,
                 w5a2_ref, w5b0_ref, w5b1_ref, w5b2_ref, a_ref, c_ref,
                 am_ref, cm_ref):
    x1 = x1_ref[...]
    x2 = x2_ref[...]
    a = _dot_t(x1, w3a_ref[...], _HI) + _dot_t(x2, w3b_ref[...], _HI)

    def shift_prev(u):
        zr = jnp.zeros((1, u.shape[1]), jnp.float32)
        return jnp.concatenate([zr, u[:-1, :]], axis=0)

    def shift_next(u):
        zr = jnp.zeros((1, u.shape[1]), jnp.float32)
        return jnp.concatenate([u[1:, :], zr], axis=0)

    c = (_dot_t(shift_prev(x1), w5a0_ref[...], _HI)
         + _dot_t(x1, w5a1_ref[...], _HI)
         + _dot_t(shift_next(x1), w5a2_ref[...], _HI)
         + _dot_t(shift_prev(x2), w5b0_ref[...], _HI)
         + _dot_t(x2, w5b1_ref[...], _HI)
         + _dot_t(shift_next(x2), w5b2_ref[...], _HI))
    a_ref[0] = a
    c_ref[0] = c
    am_ref[0] = jnp.mean(a, axis=0, keepdims=True)
    cm_ref[0] = jnp.mean(c, axis=0, keepdims=True)


def _head_b_body(g3_ref, b3_ref, sw1_ref, sw2_ref, a_ref, c_ref, am_ref,
                 cm_ref, o_ref):
    def se_scale(ym):
        y2 = jnp.maximum(_dot_t(ym, sw1_ref[...], _HI), 0.0)
        return _sigmoid(_dot_t(y2, sw2_ref[...], _HI))

    ya = se_scale(am_ref[:, 0, :])
    yc = se_scale(cm_ref[:, 0, :])
    s = a_ref[...] * ya[:, None, :] + c_ref[...] * yc[:, None, :]
    m3 = jnp.mean(s, axis=(0, 1))
    v3 = jnp.mean(s * s, axis=(0, 1)) - m3 * m3
    sc3 = g3_ref[0] * lax.rsqrt(v3 + EPS)
    u = (s - m3[None, None, :]) * sc3[None, None, :] + b3_ref[0][None, None, :]
    o_ref[...] = jnp.where(u > 0, u, 0.2 * u)


def _head(x1t, x2t, W3, W5, sw1, sw2, g3, b3, B, N):
    D1 = x1t.shape[1]
    D2 = x2t.shape[1]
    O = W3.shape[0]
    w3a, w3b = W3[:, :D1], W3[:, D1:]
    w5 = [(W5[:, :D1, t], W5[:, D1:, t]) for t in range(3)]
    wspec1 = pl.BlockSpec((O, D1), lambda b: (0, 0))
    wspec2 = pl.BlockSpec((O, D2), lambda b: (0, 0))
    a_pre, c_pre, am, cm = pl.pallas_call(
        functools.partial(_head_a_body, N),
        grid=(B,),
        in_specs=[
            pl.BlockSpec((N, D1), lambda b: (b, 0)),
            pl.BlockSpec((N, D2), lambda b: (b, 0)),
            wspec1, wspec2, wspec1, wspec1, wspec1, wspec2, wspec2, wspec2,
        ],
        out_specs=[
            pl.BlockSpec((1, N, O), lambda b: (b, 0, 0)),
            pl.BlockSpec((1, N, O), lambda b: (b, 0, 0)),
            pl.BlockSpec((1, 1, O), lambda b: (b, 0, 0)),
            pl.BlockSpec((1, 1, O), lambda b: (b, 0, 0)),
        ],
        out_shape=[
            jax.ShapeDtypeStruct((B, N, O), jnp.float32),
            jax.ShapeDtypeStruct((B, N, O), jnp.float32),
            jax.ShapeDtypeStruct((B, 1, O), jnp.float32),
            jax.ShapeDtypeStruct((B, 1, O), jnp.float32),
        ],
    )(x1t, x2t, w3a, w3b, w5[0][0], w5[1][0], w5[2][0], w5[0][1], w5[1][1],
      w5[2][1])
    out = pl.pallas_call(
        _head_b_body,
        out_shape=jax.ShapeDtypeStruct((B, N, O), jnp.float32),
    )(g3.reshape(1, O), b3.reshape(1, O), sw1, sw2, a_pre, c_pre, am, cm)
    return out


def _edge_layer(x_t, W, g, b, B, N):
    R, C = x_t.shape
    wa, wb = W[:, :C], W[:, C:]
    xpad = jnp.pad(x_t, ((0, 0), (0, 128 - C))) if C < 128 else x_t
    parts = []
    nb = B // 2
    wa_p = jnp.pad(wa, ((0, 0), (0, 128 - C))) if C < 128 else wa
    wb_p = jnp.pad(wb, ((0, 0), (0, 128 - C))) if C < 128 else wb
    for h in range(2):
        idx = _knn(x_t, N, h * nb, nb)
        E = _sc_gather(xpad, idx, N)
        xp = xpad[h * (R // 2):(h + 1) * (R // 2)]
        parts.append(_edge_pool(E, xp, wa_p, wb_p))
    mx, sm, ss = (jnp.concatenate([parts[0][i], parts[1][i]])
                  for i in range(3))
    return _combine(mx, sm, ss, g, b)


def kernel(x, xyz, W1, g1, b1, W2, g2, b2, W3, W5, g3, b3, sw1, sw2):
    B, C, N = x.shape
    x_t = jnp.transpose(x, (0, 2, 1)).reshape(B * N, C)
    x1t = _edge_layer(x_t, W1, g1, b1, B, N)
    x2t = _edge_layer(x1t, W2, g2, b2, B, N)
    out_t = _head(x1t, x2t, W3, W5, sw1, sw2, g3, b3, B, N)
    out = jnp.transpose(out_t, (0, 2, 1))
    return out, xyz


# speedup vs baseline: 1.1479x; 1.0003x over previous
import functools

import jax
import jax.numpy as jnp
from jax import lax
from jax.experimental import pallas as pl
from jax.experimental.pallas import tpu as pltpu
from jax.experimental.pallas import tpu_sc as plsc

EPS = 1e-5
KNN = 20
_CP = 32
_HI = lax.Precision.HIGHEST
_DEF = lax.Precision.DEFAULT


def _dot_t(a, b, prec):
    return lax.dot_general(a, b, (((1,), (1,)), ((), ())), precision=prec,
                           preferred_element_type=jnp.float32)


def _knn_body(N, RB, K, B0, xr_ref, xb_ref, idx_ref):
    b = pl.program_id(0) + B0
    rb = pl.program_id(1)
    xr = xr_ref[...]
    xb = xb_ref[...]
    t1 = -2.0 * _dot_t(xr, xb, _DEF)
    xxr = jnp.sum(xr * xr, axis=1)
    xx = jnp.sum(xb * xb, axis=1)
    P = (-xxr[:, None] - t1) - xx[None, :]
    iota = lax.broadcasted_iota(jnp.int32, (RB, N), 1)
    rows = lax.broadcasted_iota(jnp.int32, (RB, N), 0) + rb * RB
    off = b * N
    idx_ref[0, 0, :] = rows[:, 0] + off
    P = jnp.where(iota == rows, -1e30, P)
    for t in range(1, K):
        m = jnp.max(P, axis=1)
        cand = jnp.where(P >= m[:, None], iota, N)
        a = jnp.min(cand, axis=1)
        idx_ref[0, t, :] = a + off
        P = jnp.where(iota == a[:, None], -1e30, P)


def _knn(x_t, N, b0, nb):
    C = x_t.shape[1]
    RB = 256
    nrb = N // RB
    return pl.pallas_call(
        functools.partial(_knn_body, N, RB, KNN, b0),
        grid=(nb, nrb),
        in_specs=[
            pl.BlockSpec((RB, C), lambda b, r: ((b0 + b) * nrb + r, 0)),
            pl.BlockSpec((N, C), lambda b, r: (b0 + b, 0)),
        ],
        out_specs=pl.BlockSpec((1, KNN, RB), lambda b, r: (b, 0, r)),
        out_shape=jax.ShapeDtypeStruct((nb, KNN, N), jnp.int32),
    )(x_t, x_t)


def _sc_gather(table, idx, N):
    R = idx.shape[0] * N
    TD = table.shape[1]
    K = KNN
    info = plsc.get_sparse_core_info()
    NC, NS = info.num_cores, info.num_subcores
    NW = NC * NS
    P = R // NW
    CP = _CP
    NCH = P // CP

    mesh = plsc.VectorSubcoreMesh(core_axis_name="c", subcore_axis_name="s")

    @functools.partial(
        pl.kernel, mesh=mesh,
        out_type=jax.ShapeDtypeStruct((R * K, TD), jnp.float32),
        scratch_types=[
            pltpu.VMEM((K, P), jnp.int32),
            pltpu.VMEM((K * CP, TD), jnp.float32),
            pltpu.SemaphoreType.DMA,
        ],
    )
    def k(table_hbm, idx_hbm, e_hbm, idx_v, rows_v, sem):
        wid = lax.axis_index("s") * NC + lax.axis_index("c")
        base = wid * P
        b = base // N
        nb = base % N
        pltpu.sync_copy(idx_hbm.at[b, :, pl.ds(nb, P)], idx_v)

        def fire(c):
            return [pltpu.async_copy(
                table_hbm.at[idx_v.at[t, pl.ds(c * CP, CP)]],
                rows_v.at[pl.ds(t * CP, CP)], sem) for t in range(K)]

        for c in range(NCH):
            pend = fire(c)
            for h in pend:
                h.wait()
            row0 = (wid * NCH + c) * K * CP
            pltpu.sync_copy(rows_v, e_hbm.at[pl.ds(row0, K * CP)])

    return k(table, idx)


def _edge_pool_body(NCH, CP, K, C, e_ref, x_ref, wa_ref, wb_ref,
                    mx_ref, sm_ref, ss_ref):
    G = e_ref[...]
    xi = x_ref[...]
    xi4 = xi.reshape(NCH, 1, CP, C)
    xib = jnp.broadcast_to(xi4, (NCH, K, CP, C)).reshape(NCH * K * CP, C)
    diff = G - xib
    v = _dot_t(diff, wa_ref[...], _DEF)
    zi = _dot_t(xi, wb_ref[...], _DEF)
    O = v.shape[1]
    v4 = v.reshape(NCH, K, CP, O) + zi.reshape(NCH, 1, CP, O)
    mx_ref[...] = jnp.max(v4, axis=1).reshape(NCH * CP, O)
    sm_ref[...] = jnp.sum(v4, axis=1).reshape(NCH * CP, O)
    ss_ref[...] = jnp.sum(v4 * v4, axis=1).reshape(NCH * CP, O)


def _edge_pool(E, x_t, wa, wb):
    R, C = x_t.shape
    O = wa.shape[0]
    K = KNN
    NW = 32
    P = R // NW
    CP = _CP
    NCH = P // CP
    out_spec = pl.BlockSpec((P, O), lambda w: (w, 0))
    out_sh = jax.ShapeDtypeStruct((R, O), jnp.float32)
    return pl.pallas_call(
        functools.partial(_edge_pool_body, NCH, CP, K, C),
        grid=(NW,),
        in_specs=[
            pl.BlockSpec((P * K, E.shape[1]), lambda w: (w, 0)),
            pl.BlockSpec((P, C), lambda w: (w, 0)),
            pl.BlockSpec((O, C), lambda w: (0, 0)),
            pl.BlockSpec((O, C), lambda w: (0, 0)),
        ],
        out_specs=[out_spec, out_spec, out_spec],
        out_shape=[out_sh, out_sh, out_sh],
    )(E, x_t, wa, wb)


def _combine_body(R, K, mx_ref, sm_ref, ss_ref, g_ref, b_ref, o_ref):
    cnt = float(R * K)
    m = jnp.sum(sm_ref[...], axis=0) / cnt
    e2 = jnp.sum(ss_ref[...], axis=0) / cnt
    var = e2 - m * m
    scale = g_ref[0] * lax.rsqrt(var + EPS)
    v = (mx_ref[...] - m[None, :]) * scale[None, :] + b_ref[0][None, :]
    o_ref[...] = jnp.where(v > 0, v, 0.2 * v)


def _combine(mx, sm, ss, g, b):
    R, D = mx.shape
    return pl.pallas_call(
        functools.partial(_combine_body, R, KNN),
        out_shape=jax.ShapeDtypeStruct((R, D), jnp.float32),
    )(mx, sm, ss, g.reshape(1, D), b.reshape(1, D))


def _sigmoid(x):
    return 1.0 / (1.0 + jnp.exp(-x))


def _head_a_body(N, x1_ref, x2_ref, w3a_ref, w3b_ref, w5a0_ref, w5a1_ref,
                 w5a2_ref, w5b0_ref, w5b1_ref, w5b2_ref, a_ref, c_ref,
                 am_ref, cm_ref):
    x1 = x1_ref[...]
    x2 = x2_ref[...]
    a = _dot_t(x1, w3a_ref[...], _HI) + _dot_t(x2, w3b_ref[...], _HI)

    def shift_prev(u):
        zr = jnp.zeros((1, u.shape[1]), jnp.float32)
        return jnp.concatenate([zr, u[:-1, :]], axis=0)

    def shift_next(u):
        zr = jnp.zeros((1, u.shape[1]), jnp.float32)
        return jnp.concatenate([u[1:, :], zr], axis=0)

    c = (_dot_t(shift_prev(x1), w5a0_ref[...], _HI)
         + _dot_t(x1, w5a1_ref[...], _HI)
         + _dot_t(shift_next(x1), w5a2_ref[...], _HI)
         + _dot_t(shift_prev(x2), w5b0_ref[...], _HI)
         + _dot_t(x2, w5b1_ref[...], _HI)
         + _dot_t(shift_next(x2), w5b2_ref[...], _HI))
    a_ref[0] = a
    c_ref[0] = c
    am_ref[0] = jnp.mean(a, axis=0, keepdims=True)
    cm_ref[0] = jnp.mean(c, axis=0, keepdims=True)


def _head_b_body(g3_ref, b3_ref, sw1_ref, sw2_ref, a_ref, c_ref, am_ref,
                 cm_ref, o_ref):
    def se_scale(ym):
        y2 = jnp.maximum(_dot_t(ym, sw1_ref[...], _HI), 0.0)
        return _sigmoid(_dot_t(y2, sw2_ref[...], _HI))

    ya = se_scale(am_ref[:, 0, :])
    yc = se_scale(cm_ref[:, 0, :])
    s = a_ref[...] * ya[:, None, :] + c_ref[...] * yc[:, None, :]
    m3 = jnp.mean(s, axis=(0, 1))
    v3 = jnp.mean(s * s, axis=(0, 1)) - m3 * m3
    sc3 = g3_ref[0] * lax.rsqrt(v3 + EPS)
    u = (s - m3[None, None, :]) * sc3[None, None, :] + b3_ref[0][None, None, :]
    o_ref[...] = jnp.where(u > 0, u, 0.2 * u)


def _head(x1t, x2t, W3, W5, sw1, sw2, g3, b3, B, N):
    D1 = x1t.shape[1]
    D2 = x2t.shape[1]
    O = W3.shape[0]
    w3a, w3b = W3[:, :D1], W3[:, D1:]
    w5 = [(W5[:, :D1, t], W5[:, D1:, t]) for t in range(3)]
    wspec1 = pl.BlockSpec((O, D1), lambda b: (0, 0))
    wspec2 = pl.BlockSpec((O, D2), lambda b: (0, 0))
    a_pre, c_pre, am, cm = pl.pallas_call(
        functools.partial(_head_a_body, N),
        grid=(B,),
        in_specs=[
            pl.BlockSpec((N, D1), lambda b: (b, 0)),
            pl.BlockSpec((N, D2), lambda b: (b, 0)),
            wspec1, wspec2, wspec1, wspec1, wspec1, wspec2, wspec2, wspec2,
        ],
        out_specs=[
            pl.BlockSpec((1, N, O), lambda b: (b, 0, 0)),
            pl.BlockSpec((1, N, O), lambda b: (b, 0, 0)),
            pl.BlockSpec((1, 1, O), lambda b: (b, 0, 0)),
            pl.BlockSpec((1, 1, O), lambda b: (b, 0, 0)),
        ],
        out_shape=[
            jax.ShapeDtypeStruct((B, N, O), jnp.float32),
            jax.ShapeDtypeStruct((B, N, O), jnp.float32),
            jax.ShapeDtypeStruct((B, 1, O), jnp.float32),
            jax.ShapeDtypeStruct((B, 1, O), jnp.float32),
        ],
    )(x1t, x2t, w3a, w3b, w5[0][0], w5[1][0], w5[2][0], w5[0][1], w5[1][1],
      w5[2][1])
    out = pl.pallas_call(
        _head_b_body,
        out_shape=jax.ShapeDtypeStruct((B, N, O), jnp.float32),
    )(g3.reshape(1, O), b3.reshape(1, O), sw1, sw2, a_pre, c_pre, am, cm)
    return out


def _edge_layer(x_t, W, g, b, B, N):
    R, C = x_t.shape
    wa, wb = W[:, :C], W[:, C:]
    xpad = jnp.pad(x_t, ((0, 0), (0, 128 - C))) if C < 128 else x_t
    parts = []
    nb = B // 2
    wa_p = jnp.pad(wa, ((0, 0), (0, 128 - C))) if C < 128 else wa
    wb_p = jnp.pad(wb, ((0, 0), (0, 128 - C))) if C < 128 else wb
    for h in range(2):
        idx = _knn(x_t, N, h * nb, nb)
        E = _sc_gather(xpad, idx, N)
        xp = xpad[h * (R // 2):(h + 1) * (R // 2)]
        parts.append(_edge_pool(E, xp, wa_p, wb_p))
    mx, sm, ss = (jnp.concatenate([parts[0][i], parts[1][i]])
                  for i in range(3))
    return _combine(mx, sm, ss, g, b)


def kernel(x, xyz, W1, g1, b1, W2, g2, b2, W3, W5, g3, b3, sw1, sw2):
    B, C, N = x.shape
    x_t = jnp.transpose(x, (0, 2, 1)).reshape(B * N, C)
    x1t = _edge_layer(x_t, W1, g1, b1, B, N)
    x2t = _edge_layer(x1t, W2, g2, b2, B, N)
    out_t = _head(x1t, x2t, W3, W5, sw1, sw2, g3, b3, B, N)
    out = jnp.transpose(out_t, (0, 2, 1))
    return out, xyz
